# baseline (speedup 1.0000x reference)
import functools
import jax
import jax.numpy as jnp
from jax import lax
from jax.experimental import pallas as pl
from jax.experimental.pallas import tpu as pltpu
from jax.experimental.pallas import tpu_sc as plsc

_N = 100000
_NP = 100352
_E = 3200000
_LANES = 128
_ROWS = _E // _LANES
_K = 8
_NBLK = _ROWS // _K
_PAIRS = 48
_SL = _NP // 16
_CH = 448
_NCH = _SL // _CH
_mesh = plsc.VectorSubcoreMesh(core_axis_name="c", subcore_axis_name="s")


def _edge_split(wid):
    b0 = wid * _NBLK // 32
    b1 = (wid + 1) * _NBLK // 32
    return b0, b1 - b0


@functools.partial(
    pl.kernel,
    out_type=[
        jax.ShapeDtypeStruct((_NP, 8), jnp.float32),
        jax.ShapeDtypeStruct((_NP, 8), jnp.float32),
    ],
    mesh=_mesh,
    scratch_types=[
        pltpu.VMEM((2, _K, _LANES), jnp.int32),
        pltpu.VMEM((2, _K, _LANES), jnp.int32),
        pltpu.VMEM((2, _K, _LANES, 8), jnp.float32),
        pltpu.VMEM_SHARED((_NP, 8), jnp.float32),
        pltpu.SemaphoreType.DMA,
        pltpu.SemaphoreType.DMA,
        pltpu.SemaphoreType.DMA,
        pltpu.SemaphoreType.DMA,
        pltpu.SemaphoreType.DMA,
        pltpu.SemaphoreType.DMA,
    ],
    compiler_params=pltpu.CompilerParams(use_tc_tiling_on_sc=False),
)
def _edge_pass1(x5, src2d, dst2d, zeros5, agg0, agg1,
                idx_s, idx_d, rows, acc,
                si0, si1, sg0, sg1, ss0, ss1):
    c = lax.axis_index("c")
    s = lax.axis_index("s")
    wid = s * 2 + c
    b0, nblk = _edge_split(wid)
    si = (si0, si1)
    sg = (sg0, sg1)
    ss = (ss0, ss1)

    def load_idx(b, blk, sem):
        r0 = (b0 + blk) * _K
        pltpu.async_copy(src2d.at[pl.ds(r0, _K)], idx_s.at[b], sem)
        pltpu.async_copy(dst2d.at[pl.ds(r0, _K)], idx_d.at[b], sem)

    def wait_idx(b, sem):
        pltpu.make_async_copy(src2d.at[pl.ds(0, _K)], idx_s.at[b], sem).wait()
        pltpu.make_async_copy(dst2d.at[pl.ds(0, _K)], idx_d.at[b], sem).wait()

    def fire_gathers(b, sem):
        for j in range(_K):
            pltpu.async_copy(x5.at[idx_s.at[b].at[j]], rows.at[b].at[j], sem)

    def wait_gathers(b, sem):
        for j in range(_K):
            pltpu.make_async_copy(
                x5.at[idx_s.at[b].at[j]], rows.at[b].at[j], sem).wait()

    def fire_scatters(b, sem):
        for j in range(_K):
            pltpu.async_copy(
                rows.at[b].at[j], acc.at[idx_d.at[b].at[j]], sem, add=True)

    def wait_scatters(b, sem):
        for j in range(_K):
            pltpu.make_async_copy(
                rows.at[b].at[j], acc.at[idx_d.at[b].at[j]], sem).wait()

    ns = s * _SL
    pltpu.sync_copy(zeros5.at[pl.ds(ns, _SL)], acc.at[pl.ds(ns, _SL)])
    plsc.subcore_barrier()

    load_idx(0, 0, si[0])

    def pair(p, carry):
        for b in (0, 1):
            i = 2 * p + b
            wait_idx(b, si[b])
            fire_gathers(b, sg[b])
            if b == 0:
                @pl.when(p > 0)
                def _():
                    wait_scatters(1, ss[1])
            else:
                wait_scatters(0, ss[0])
            load_idx(1 - b, i + 1, si[1 - b])
            wait_gathers(b, sg[b])
            fire_scatters(b, ss[b])
        return carry

    lax.fori_loop(0, _PAIRS, pair, 0)

    wait_idx(0, si[0])
    fire_gathers(0, sg[0])
    wait_scatters(1, ss[1])

    @pl.when(nblk > 97)
    def _():
        load_idx(1, 97, si[1])
    wait_gathers(0, sg[0])
    fire_scatters(0, ss[0])

    @pl.when(nblk > 97)
    def _():
        wait_idx(1, si[1])
        fire_gathers(1, sg[1])
        wait_gathers(1, sg[1])
        fire_scatters(1, ss[1])
    wait_scatters(0, ss[0])

    @pl.when(nblk > 97)
    def _():
        wait_scatters(1, ss[1])

    plsc.subcore_barrier()

    @pl.when(c == 0)
    def _():
        pltpu.sync_copy(acc.at[pl.ds(ns, _SL)], agg0.at[pl.ds(ns, _SL)])

    @pl.when(c == 1)
    def _():
        pltpu.sync_copy(acc.at[pl.ds(ns, _SL)], agg1.at[pl.ds(ns, _SL)])


@functools.partial(
    pl.kernel,
    out_type=[
        jax.ShapeDtypeStruct((_NP,), jnp.float32),
        jax.ShapeDtypeStruct((_NP,), jnp.float32),
        jax.ShapeDtypeStruct((_NP,), jnp.float32),
        jax.ShapeDtypeStruct((_NP,), jnp.float32),
    ],
    mesh=_mesh,
    scratch_types=[
        pltpu.VMEM((_NP // 2,), jnp.float32),
        pltpu.VMEM((2, _K, _LANES), jnp.int32),
        pltpu.VMEM((2, _K, _LANES), jnp.int32),
        pltpu.VMEM((2, _K, _LANES), jnp.float32),
        pltpu.VMEM((_CH, 8), jnp.float32),
        pltpu.VMEM((_CH, 8), jnp.float32),
        pltpu.VMEM((_CH, 8), jnp.float32),
        pltpu.VMEM((_CH,), jnp.float32),
        pltpu.VMEM((_CH,), jnp.float32),
        pltpu.VMEM((_CH,), jnp.float32),
        pltpu.VMEM((192,), jnp.float32),
        pltpu.VMEM_SHARED((_NP,), jnp.float32),
        pltpu.VMEM_SHARED((_NP,), jnp.float32),
        pltpu.SemaphoreType.DMA,
        pltpu.SemaphoreType.DMA,
        pltpu.SemaphoreType.DMA,
        pltpu.SemaphoreType.DMA,
    ],
    compiler_params=pltpu.CompilerParams(
        needs_layout_passes=False, use_tc_tiling_on_sc=False),
)
def _pass2(agg0, agg1, x5, src2d, dst2d, wpack,
           a2o0, a2o1, v_out, inv_out,
           u_v, idx_s, idx_d, vals, p0, p1, xs, us, vs, invs, wv,
           u_sh, acc, si0, si1, ss0, ss1):
    c = lax.axis_index("c")
    s = lax.axis_index("s")
    wid = s * 2 + c
    b0, nblk = _edge_split(wid)
    si = (si0, si1)
    ss = (ss0, ss1)
    ns = s * _SL

    pltpu.sync_copy(wpack, wv)
    for z in range(_CH // 16):
        us[pl.ds(z * 16, 16)] = jnp.zeros((16,), jnp.float32)
    for ci in range(_NCH):
        pltpu.sync_copy(us, acc.at[pl.ds(ns + ci * _CH, _CH)])
    lanes16 = lax.iota(jnp.int32, 16)
    wrow = [wv[pl.ds(r * 16, 16)] for r in range(12)]

    def chunk_body(ci, carry):
        base = ns + ci * _CH
        pltpu.sync_copy(agg0.at[pl.ds(base, _CH)], p0)
        pltpu.sync_copy(agg1.at[pl.ds(base, _CH)], p1)
        pltpu.sync_copy(x5.at[pl.ds(base, _CH)], xs)

        def group_body(g, carry2):
            r = g * 16 + lanes16
            k4 = jnp.full((16,), 4, jnp.int32)
            cv = plsc.load_gather(p0, [r, k4]) + plsc.load_gather(p1, [r, k4])
            inv = 1.0 / jnp.maximum(cv, 1.0)
            mean = []
            xk = []
            for k in range(4):
                kk = jnp.full((16,), k, jnp.int32)
                ak = (plsc.load_gather(p0, [r, kk])
                      + plsc.load_gather(p1, [r, kk]))
                mean.append(ak * inv)
                xk.append(plsc.load_gather(xs, [r, kk]))
            uacc = jnp.zeros((16,), jnp.float32)
            vacc = jnp.zeros((16,), jnp.float32)
            for j in range(16):
                t = jnp.broadcast_to(wrow[8][j], (16,))
                for k in range(4):
                    t = t + mean[k] * wrow[k][j] + xk[k] * wrow[4 + k][j]
                h = jnp.maximum(t, 0.0)
                uacc = uacc + h * wrow[9][j]
                vacc = vacc + h * wrow[10][j]
            us[pl.ds(g * 16, 16)] = uacc
            vs[pl.ds(g * 16, 16)] = vacc + wrow[11][0]
            invs[pl.ds(g * 16, 16)] = inv
            return carry2

        lax.fori_loop(0, _CH // 16, group_body, 0)
        pltpu.sync_copy(us, u_sh.at[pl.ds(base, _CH)])
        pltpu.sync_copy(vs, v_out.at[pl.ds(base, _CH)])
        pltpu.sync_copy(invs, inv_out.at[pl.ds(base, _CH)])
        return carry

    lax.fori_loop(0, _NCH, chunk_body, 0)
    plsc.subcore_barrier()

    def load_idx(b, blk, sem):
        r0 = (b0 + blk) * _K
        pltpu.async_copy(src2d.at[pl.ds(r0, _K)], idx_s.at[b], sem)
        pltpu.async_copy(dst2d.at[pl.ds(r0, _K)], idx_d.at[b], sem)

    def wait_idx(b, sem):
        pltpu.make_async_copy(src2d.at[pl.ds(0, _K)], idx_s.at[b], sem).wait()
        pltpu.make_async_copy(dst2d.at[pl.ds(0, _K)], idx_d.at[b], sem).wait()

    half = _NP // 2

    def compute(b, off):
        for j in range(_K):
            row = idx_s.at[b].at[j]
            vrow = vals.at[b].at[j]
            for k in range(_LANES // 16):
                ii = row[pl.ds(k * 16, 16)] - off
                msk = (ii >= 0) & (ii < half)
                iic = jnp.clip(ii, 0, half - 1)
                g = plsc.load_gather(u_v, [iic])
                vrow[pl.ds(k * 16, 16)] = jnp.where(msk, g, 0.0)

    def fire_scatters(b, sem):
        for j in range(_K):
            pltpu.async_copy(
                vals.at[b].at[j], acc.at[idx_d.at[b].at[j]], sem, add=True)

    def wait_scatters(b, sem):
        for j in range(_K):
            pltpu.make_async_copy(
                vals.at[b].at[j], acc.at[idx_d.at[b].at[j]], sem).wait()

    for h in range(2):
        off = h * half
        pltpu.sync_copy(u_sh.at[pl.ds(off, half)], u_v)
        load_idx(0, 0, si[0])

        def pair(p, carry, _off=off):
            for b in (0, 1):
                i = 2 * p + b
                wait_idx(b, si[b])
                compute(b, _off)
                if b == 0:
                    @pl.when(p > 0)
                    def _():
                        wait_scatters(1, ss[1])
                else:
                    wait_scatters(0, ss[0])
                fire_scatters(b, ss[b])
                load_idx(1 - b, i + 1, si[1 - b])
            return carry

        lax.fori_loop(0, _PAIRS, pair, 0)

        wait_idx(0, si[0])
        compute(0, off)
        wait_scatters(1, ss[1])
        fire_scatters(0, ss[0])

        @pl.when(nblk > 97)
        def _():
            load_idx(1, 97, si[1])
            wait_idx(1, si[1])
            compute(1, off)
            wait_scatters(0, ss[0])
            fire_scatters(1, ss[1])
            wait_scatters(1, ss[1])

        @pl.when(nblk <= 97)
        def _():
            wait_scatters(0, ss[0])

    plsc.subcore_barrier()

    for ci in range(_NCH):
        sl_ci = pl.ds(ns + ci * _CH, _CH)
        pltpu.sync_copy(acc.at[sl_ci], us)

        @pl.when(c == 0)
        def _():
            pltpu.sync_copy(us, a2o0.at[sl_ci])

        @pl.when(c == 1)
        def _():
            pltpu.sync_copy(us, a2o1.at[sl_ci])


@functools.partial(
    pl.kernel,
    out_type=jax.ShapeDtypeStruct((_NP,), jnp.float32),
    mesh=_mesh,
    scratch_types=[
        pltpu.VMEM((_SL,), jnp.float32),
        pltpu.VMEM((_SL,), jnp.float32),
        pltpu.VMEM((_SL,), jnp.float32),
        pltpu.VMEM((_SL,), jnp.float32),
        pltpu.VMEM((_SL,), jnp.float32),
    ],
)
def _final(a2_0, a2_1, inv, v, out, a0, a1, iv, vv, ob):
    c = lax.axis_index("c")
    s = lax.axis_index("s")
    sl = s * 2 + c
    ns = sl * _SL

    @pl.when(sl < 16)
    def _():
        pltpu.sync_copy(a2_0.at[pl.ds(ns, _SL)], a0)
        pltpu.sync_copy(a2_1.at[pl.ds(ns, _SL)], a1)
        pltpu.sync_copy(inv.at[pl.ds(ns, _SL)], iv)
        pltpu.sync_copy(v.at[pl.ds(ns, _SL)], vv)

        def body(g, carry):
            sl16 = pl.ds(g * 16, 16)
            ob[sl16] = (a0[sl16] + a1[sl16]) * iv[sl16] + vv[sl16]
            return carry

        lax.fori_loop(0, _SL // 16, body, 0)
        pltpu.sync_copy(ob, out.at[pl.ds(ns, _SL)])


def kernel(x, edge_index, W1l, b1, W1r, W2l, b2, W2r):
    f32 = jnp.float32
    e3 = edge_index.reshape(2, _ROWS, _LANES)
    wpack = jnp.concatenate([
        W1l.T,
        W1r.T,
        b1.reshape(1, 16),
        W2l.reshape(1, 16),
        W2r.reshape(1, 16),
        jnp.broadcast_to(b2.reshape(1, 1), (1, 16)),
    ]).astype(f32).reshape(192)
    x5 = jnp.pad(jnp.concatenate([x, jnp.ones((_N, 1), f32)], axis=1),
                 ((0, _NP - _N), (0, 3)))
    zeros5 = jnp.zeros((_NP, 8), f32)

    src2d = e3[0]
    dst2d = e3[1]
    agg0, agg1 = _edge_pass1(x5, src2d, dst2d, zeros5)
    a2_0, a2_1, v, inv = _pass2(agg0, agg1, x5, src2d, dst2d, wpack)
    out = _final(a2_0, a2_1, inv, v)
    return out[:_N].reshape(_N, 1)


# speedup vs baseline: 1.1760x; 1.1760x over previous
import functools
import jax
import jax.numpy as jnp
from jax import lax
from jax.experimental import pallas as pl
from jax.experimental.pallas import tpu as pltpu
from jax.experimental.pallas import tpu_sc as plsc

_N = 100000
_NP = 100352
_E = 3200000
_LANES = 128
_ROWS = _E // _LANES
_K = 8
_NBLK = _ROWS // _K
_PAIRS = 48
_SL = _NP // 16
_CH = 448
_NCH = _SL // _CH
_mesh = plsc.VectorSubcoreMesh(core_axis_name="c", subcore_axis_name="s")


def _edge_split(wid):
    b0 = wid * _NBLK // 32
    b1 = (wid + 1) * _NBLK // 32
    return b0, b1 - b0


@functools.partial(
    pl.kernel,
    out_type=[
        jax.ShapeDtypeStruct((_NP, 8), jnp.float32),
        jax.ShapeDtypeStruct((_NP, 8), jnp.float32),
    ],
    mesh=_mesh,
    scratch_types=[
        pltpu.VMEM((2, _K, _LANES), jnp.int32),
        pltpu.VMEM((2, _K, _LANES), jnp.int32),
        pltpu.VMEM((2, _K, _LANES, 8), jnp.float32),
        pltpu.VMEM_SHARED((_NP, 8), jnp.float32),
        pltpu.SemaphoreType.DMA,
        pltpu.SemaphoreType.DMA,
        pltpu.SemaphoreType.DMA,
        pltpu.SemaphoreType.DMA,
        pltpu.SemaphoreType.DMA,
        pltpu.SemaphoreType.DMA,
    ],
    compiler_params=pltpu.CompilerParams(use_tc_tiling_on_sc=False),
)
def _edge_pass1(x5, src2d, dst2d, zeros5, agg0, agg1,
                idx_s, idx_d, rows, acc,
                si0, si1, sg0, sg1, ss0, ss1):
    c = lax.axis_index("c")
    s = lax.axis_index("s")
    wid = s * 2 + c
    b0, nblk = _edge_split(wid)
    si = (si0, si1)
    sg = (sg0, sg1)
    ss = (ss0, ss1)

    def load_idx(b, blk, sem):
        r0 = (b0 + blk) * _K
        pltpu.async_copy(src2d.at[pl.ds(r0, _K)], idx_s.at[b], sem)
        pltpu.async_copy(dst2d.at[pl.ds(r0, _K)], idx_d.at[b], sem)

    def wait_idx(b, sem):
        pltpu.make_async_copy(src2d.at[pl.ds(0, _K)], idx_s.at[b], sem).wait()
        pltpu.make_async_copy(dst2d.at[pl.ds(0, _K)], idx_d.at[b], sem).wait()

    def fire_gathers(b, sem):
        for j in range(_K):
            pltpu.async_copy(x5.at[idx_s.at[b].at[j]], rows.at[b].at[j], sem)

    def wait_gathers(b, sem):
        for j in range(_K):
            pltpu.make_async_copy(
                x5.at[idx_s.at[b].at[j]], rows.at[b].at[j], sem).wait()

    def fire_scatters(b, sem):
        for j in range(_K):
            pltpu.async_copy(
                rows.at[b].at[j], acc.at[idx_d.at[b].at[j]], sem, add=True)

    def wait_scatters(b, sem):
        for j in range(_K):
            pltpu.make_async_copy(
                rows.at[b].at[j], acc.at[idx_d.at[b].at[j]], sem).wait()

    ns = s * _SL
    pltpu.sync_copy(zeros5.at[pl.ds(ns, _SL)], acc.at[pl.ds(ns, _SL)])
    plsc.subcore_barrier()

    load_idx(0, 0, si[0])

    def pair(p, carry):
        for b in (0, 1):
            i = 2 * p + b
            wait_idx(b, si[b])
            fire_gathers(b, sg[b])
            if b == 0:
                @pl.when(p > 0)
                def _():
                    wait_scatters(1, ss[1])
            else:
                wait_scatters(0, ss[0])
            load_idx(1 - b, i + 1, si[1 - b])
            wait_gathers(b, sg[b])
            fire_scatters(b, ss[b])
        return carry

    lax.fori_loop(0, _PAIRS, pair, 0)

    wait_idx(0, si[0])
    fire_gathers(0, sg[0])
    wait_scatters(1, ss[1])

    @pl.when(nblk > 97)
    def _():
        load_idx(1, 97, si[1])
    wait_gathers(0, sg[0])
    fire_scatters(0, ss[0])

    @pl.when(nblk > 97)
    def _():
        wait_idx(1, si[1])
        fire_gathers(1, sg[1])
        wait_gathers(1, sg[1])
        fire_scatters(1, ss[1])
    wait_scatters(0, ss[0])

    @pl.when(nblk > 97)
    def _():
        wait_scatters(1, ss[1])

    plsc.subcore_barrier()

    @pl.when(c == 0)
    def _():
        pltpu.sync_copy(acc.at[pl.ds(ns, _SL)], agg0.at[pl.ds(ns, _SL)])

    @pl.when(c == 1)
    def _():
        pltpu.sync_copy(acc.at[pl.ds(ns, _SL)], agg1.at[pl.ds(ns, _SL)])


@functools.partial(
    pl.kernel,
    out_type=[
        jax.ShapeDtypeStruct((_NP,), jnp.float32),
        jax.ShapeDtypeStruct((_NP,), jnp.float32),
        jax.ShapeDtypeStruct((_NP,), jnp.float32),
        jax.ShapeDtypeStruct((_NP,), jnp.float32),
        jax.ShapeDtypeStruct((_NP,), jnp.float32),
    ],
    mesh=_mesh,
    scratch_types=[
        pltpu.VMEM((_NP,), jnp.float32),
        pltpu.VMEM((2, _K, _LANES), jnp.int32),
        pltpu.VMEM((2, _K, _LANES), jnp.int32),
        pltpu.VMEM((2, _K, _LANES), jnp.float32),
        pltpu.VMEM((_CH, 8), jnp.float32),
        pltpu.VMEM((_CH, 8), jnp.float32),
        pltpu.VMEM((_CH, 8), jnp.float32),
        pltpu.VMEM((_CH,), jnp.float32),
        pltpu.VMEM((_CH,), jnp.float32),
        pltpu.VMEM((_CH,), jnp.float32),
        pltpu.VMEM((192,), jnp.float32),
        pltpu.VMEM_SHARED((_NP,), jnp.float32),
        pltpu.SemaphoreType.DMA,
        pltpu.SemaphoreType.DMA,
        pltpu.SemaphoreType.DMA,
        pltpu.SemaphoreType.DMA,
    ],
    compiler_params=pltpu.CompilerParams(
        needs_layout_passes=False, use_tc_tiling_on_sc=False),
)
def _pass2(agg0, agg1, x5, src2d, dst2d, wpack,
           a2o0, a2o1, v_out, inv_out, u_out,
           u_v, idx_s, idx_d, vals, p0, p1, xs, us, vs, invs, wv,
           acc, si0, si1, ss0, ss1):
    c = lax.axis_index("c")
    s = lax.axis_index("s")
    wid = s * 2 + c
    b0, nblk = _edge_split(wid)
    si = (si0, si1)
    ss = (ss0, ss1)
    ns = s * _SL

    pltpu.sync_copy(wpack, wv)
    for z in range(_CH // 16):
        us[pl.ds(z * 16, 16)] = jnp.zeros((16,), jnp.float32)
    for ci in range(_NCH):
        pltpu.sync_copy(us, acc.at[pl.ds(ns + ci * _CH, _CH)])
    lanes16 = lax.iota(jnp.int32, 16)
    wrow = [wv[pl.ds(r * 16, 16)] for r in range(12)]

    def chunk_body(ci, carry):
        base = ns + ci * _CH
        pltpu.sync_copy(agg0.at[pl.ds(base, _CH)], p0)
        pltpu.sync_copy(agg1.at[pl.ds(base, _CH)], p1)
        pltpu.sync_copy(x5.at[pl.ds(base, _CH)], xs)

        def group_body(g, carry2):
            r = g * 16 + lanes16
            k4 = jnp.full((16,), 4, jnp.int32)
            cv = plsc.load_gather(p0, [r, k4]) + plsc.load_gather(p1, [r, k4])
            inv = 1.0 / jnp.maximum(cv, 1.0)
            mean = []
            xk = []
            for k in range(4):
                kk = jnp.full((16,), k, jnp.int32)
                ak = (plsc.load_gather(p0, [r, kk])
                      + plsc.load_gather(p1, [r, kk]))
                mean.append(ak * inv)
                xk.append(plsc.load_gather(xs, [r, kk]))
            uacc = jnp.zeros((16,), jnp.float32)
            vacc = jnp.zeros((16,), jnp.float32)
            for j in range(16):
                t = jnp.broadcast_to(wrow[8][j], (16,))
                for k in range(4):
                    t = t + mean[k] * wrow[k][j] + xk[k] * wrow[4 + k][j]
                h = jnp.maximum(t, 0.0)
                uacc = uacc + h * wrow[9][j]
                vacc = vacc + h * wrow[10][j]
            us[pl.ds(g * 16, 16)] = uacc
            vs[pl.ds(g * 16, 16)] = vacc + wrow[11][0]
            invs[pl.ds(g * 16, 16)] = inv
            return carry2

        lax.fori_loop(0, _CH // 16, group_body, 0)
        pltpu.sync_copy(us, u_out.at[pl.ds(base, _CH)])
        pltpu.sync_copy(vs, v_out.at[pl.ds(base, _CH)])
        pltpu.sync_copy(invs, inv_out.at[pl.ds(base, _CH)])
        return carry

    lax.fori_loop(0, _NCH, chunk_body, 0)
    plsc.subcore_barrier()
    pltpu.sync_copy(u_out, u_v)

    def load_idx(b, blk, sem):
        r0 = (b0 + blk) * _K
        pltpu.async_copy(src2d.at[pl.ds(r0, _K)], idx_s.at[b], sem)
        pltpu.async_copy(dst2d.at[pl.ds(r0, _K)], idx_d.at[b], sem)

    def wait_idx(b, sem):
        pltpu.make_async_copy(src2d.at[pl.ds(0, _K)], idx_s.at[b], sem).wait()
        pltpu.make_async_copy(dst2d.at[pl.ds(0, _K)], idx_d.at[b], sem).wait()

    def compute(b):
        for j in range(_K):
            row = idx_s.at[b].at[j]
            vrow = vals.at[b].at[j]
            for k in range(_LANES // 16):
                ii = row[pl.ds(k * 16, 16)]
                vrow[pl.ds(k * 16, 16)] = plsc.load_gather(u_v, [ii])

    def fire_scatters(b, sem):
        for j in range(_K):
            pltpu.async_copy(
                vals.at[b].at[j], acc.at[idx_d.at[b].at[j]], sem, add=True)

    def wait_scatters(b, sem):
        for j in range(_K):
            pltpu.make_async_copy(
                vals.at[b].at[j], acc.at[idx_d.at[b].at[j]], sem).wait()

    load_idx(0, 0, si[0])

    def pair(p, carry):
        for b in (0, 1):
            i = 2 * p + b
            wait_idx(b, si[b])
            compute(b)
            if b == 0:
                @pl.when(p > 0)
                def _():
                    wait_scatters(1, ss[1])
            else:
                wait_scatters(0, ss[0])
            fire_scatters(b, ss[b])
            load_idx(1 - b, i + 1, si[1 - b])
        return carry

    lax.fori_loop(0, _PAIRS, pair, 0)

    wait_idx(0, si[0])
    compute(0)
    wait_scatters(1, ss[1])
    fire_scatters(0, ss[0])

    @pl.when(nblk > 97)
    def _():
        load_idx(1, 97, si[1])
        wait_idx(1, si[1])
        compute(1)
        wait_scatters(0, ss[0])
        fire_scatters(1, ss[1])
        wait_scatters(1, ss[1])

    @pl.when(nblk <= 97)
    def _():
        wait_scatters(0, ss[0])

    plsc.subcore_barrier()

    for ci in range(_NCH):
        sl_ci = pl.ds(ns + ci * _CH, _CH)
        pltpu.sync_copy(acc.at[sl_ci], us)

        @pl.when(c == 0)
        def _():
            pltpu.sync_copy(us, a2o0.at[sl_ci])

        @pl.when(c == 1)
        def _():
            pltpu.sync_copy(us, a2o1.at[sl_ci])


@functools.partial(
    pl.kernel,
    out_type=jax.ShapeDtypeStruct((_NP,), jnp.float32),
    mesh=_mesh,
    scratch_types=[
        pltpu.VMEM((_SL,), jnp.float32),
        pltpu.VMEM((_SL,), jnp.float32),
        pltpu.VMEM((_SL,), jnp.float32),
        pltpu.VMEM((_SL,), jnp.float32),
        pltpu.VMEM((_SL,), jnp.float32),
    ],
)
def _final(a2_0, a2_1, inv, v, out, a0, a1, iv, vv, ob):
    c = lax.axis_index("c")
    s = lax.axis_index("s")
    sl = s * 2 + c
    ns = sl * _SL

    @pl.when(sl < 16)
    def _():
        pltpu.sync_copy(a2_0.at[pl.ds(ns, _SL)], a0)
        pltpu.sync_copy(a2_1.at[pl.ds(ns, _SL)], a1)
        pltpu.sync_copy(inv.at[pl.ds(ns, _SL)], iv)
        pltpu.sync_copy(v.at[pl.ds(ns, _SL)], vv)

        def body(g, carry):
            sl16 = pl.ds(g * 16, 16)
            ob[sl16] = (a0[sl16] + a1[sl16]) * iv[sl16] + vv[sl16]
            return carry

        lax.fori_loop(0, _SL // 16, body, 0)
        pltpu.sync_copy(ob, out.at[pl.ds(ns, _SL)])


def kernel(x, edge_index, W1l, b1, W1r, W2l, b2, W2r):
    f32 = jnp.float32
    e3 = edge_index.reshape(2, _ROWS, _LANES)
    wpack = jnp.concatenate([
        W1l.T,
        W1r.T,
        b1.reshape(1, 16),
        W2l.reshape(1, 16),
        W2r.reshape(1, 16),
        jnp.broadcast_to(b2.reshape(1, 1), (1, 16)),
    ]).astype(f32).reshape(192)
    x5 = jnp.pad(jnp.concatenate([x, jnp.ones((_N, 1), f32)], axis=1),
                 ((0, _NP - _N), (0, 3)))
    zeros5 = jnp.zeros((_NP, 8), f32)

    src2d = e3[0]
    dst2d = e3[1]
    agg0, agg1 = _edge_pass1(x5, src2d, dst2d, zeros5)
    a2_0, a2_1, v, inv, _u = _pass2(agg0, agg1, x5, src2d, dst2d, wpack)
    out = _final(a2_0, a2_1, inv, v)
    return out[:_N].reshape(_N, 1)


# speedup vs baseline: 1.2225x; 1.0395x over previous
import functools
import jax
import jax.numpy as jnp
from jax import lax
from jax.experimental import pallas as pl
from jax.experimental.pallas import tpu as pltpu
from jax.experimental.pallas import tpu_sc as plsc

_N = 100000
_NP = 100352
_E = 3200000
_LANES = 128
_ROWS = _E // _LANES
_K = 8
_NBLK = _ROWS // _K
_PAIRS = 48
_SL = _NP // 16
_CH = 448
_NCH = _SL // _CH
_mesh = plsc.VectorSubcoreMesh(core_axis_name="c", subcore_axis_name="s")


def _edge_split(wid):
    b0 = wid * _NBLK // 32
    b1 = (wid + 1) * _NBLK // 32
    return b0, b1 - b0


@functools.partial(
    pl.kernel,
    out_type=[
        jax.ShapeDtypeStruct((_NP, 8), jnp.float32),
        jax.ShapeDtypeStruct((_NP, 8), jnp.float32),
    ],
    mesh=_mesh,
    scratch_types=[
        pltpu.VMEM((2, _K, _LANES), jnp.int32),
        pltpu.VMEM((2, _K, _LANES), jnp.int32),
        pltpu.VMEM((2, _K, _LANES, 8), jnp.float32),
        pltpu.VMEM_SHARED((_NP, 8), jnp.float32),
        pltpu.SemaphoreType.DMA,
        pltpu.SemaphoreType.DMA,
        pltpu.SemaphoreType.DMA,
        pltpu.SemaphoreType.DMA,
        pltpu.SemaphoreType.DMA,
        pltpu.SemaphoreType.DMA,
    ],
    compiler_params=pltpu.CompilerParams(use_tc_tiling_on_sc=False),
)
def _edge_pass1(x5, e3, zeros5, agg0, agg1,
                idx_s, idx_d, rows, acc,
                si0, si1, sg0, sg1, ss0, ss1):
    c = lax.axis_index("c")
    s = lax.axis_index("s")
    wid = s * 2 + c
    b0, nblk = _edge_split(wid)
    si = (si0, si1)
    sg = (sg0, sg1)
    ss = (ss0, ss1)

    def load_idx(b, blk, sem):
        r0 = (b0 + blk) * _K
        pltpu.async_copy(e3.at[0, pl.ds(r0, _K)], idx_s.at[b], sem)
        pltpu.async_copy(e3.at[1, pl.ds(r0, _K)], idx_d.at[b], sem)

    def wait_idx(b, sem):
        pltpu.make_async_copy(e3.at[0, pl.ds(0, _K)], idx_s.at[b], sem).wait()
        pltpu.make_async_copy(e3.at[1, pl.ds(0, _K)], idx_d.at[b], sem).wait()

    def fire_gathers(b, sem):
        for j in range(_K):
            pltpu.async_copy(x5.at[idx_s.at[b].at[j]], rows.at[b].at[j], sem)

    def wait_gathers(b, sem):
        for j in range(_K):
            pltpu.make_async_copy(
                x5.at[idx_s.at[b].at[j]], rows.at[b].at[j], sem).wait()

    def fire_scatters(b, sem):
        for j in range(_K):
            pltpu.async_copy(
                rows.at[b].at[j], acc.at[idx_d.at[b].at[j]], sem, add=True)

    def wait_scatters(b, sem):
        for j in range(_K):
            pltpu.make_async_copy(
                rows.at[b].at[j], acc.at[idx_d.at[b].at[j]], sem).wait()

    ns = s * _SL
    pltpu.sync_copy(zeros5.at[pl.ds(ns, _SL)], acc.at[pl.ds(ns, _SL)])
    plsc.subcore_barrier()

    load_idx(0, 0, si[0])

    def pair(p, carry):
        for b in (0, 1):
            i = 2 * p + b
            wait_idx(b, si[b])
            fire_gathers(b, sg[b])
            if b == 0:
                @pl.when(p > 0)
                def _():
                    wait_scatters(1, ss[1])
            else:
                wait_scatters(0, ss[0])
            load_idx(1 - b, i + 1, si[1 - b])
            wait_gathers(b, sg[b])
            fire_scatters(b, ss[b])
        return carry

    lax.fori_loop(0, _PAIRS, pair, 0)

    wait_idx(0, si[0])
    fire_gathers(0, sg[0])
    wait_scatters(1, ss[1])

    @pl.when(nblk > 97)
    def _():
        load_idx(1, 97, si[1])
    wait_gathers(0, sg[0])
    fire_scatters(0, ss[0])

    @pl.when(nblk > 97)
    def _():
        wait_idx(1, si[1])
        fire_gathers(1, sg[1])
        wait_gathers(1, sg[1])
        fire_scatters(1, ss[1])
    wait_scatters(0, ss[0])

    @pl.when(nblk > 97)
    def _():
        wait_scatters(1, ss[1])

    plsc.subcore_barrier()

    @pl.when(c == 0)
    def _():
        pltpu.sync_copy(acc.at[pl.ds(ns, _SL)], agg0.at[pl.ds(ns, _SL)])

    @pl.when(c == 1)
    def _():
        pltpu.sync_copy(acc.at[pl.ds(ns, _SL)], agg1.at[pl.ds(ns, _SL)])


@functools.partial(
    pl.kernel,
    out_type=[
        jax.ShapeDtypeStruct((_NP,), jnp.float32),
        jax.ShapeDtypeStruct((_NP,), jnp.float32),
        jax.ShapeDtypeStruct((_NP,), jnp.float32),
        jax.ShapeDtypeStruct((_NP,), jnp.float32),
        jax.ShapeDtypeStruct((_NP,), jnp.float32),
    ],
    mesh=_mesh,
    scratch_types=[
        pltpu.VMEM((_NP,), jnp.float32),
        pltpu.VMEM((2, _K, _LANES), jnp.int32),
        pltpu.VMEM((2, _K, _LANES), jnp.int32),
        pltpu.VMEM((2, _K, _LANES), jnp.float32),
        pltpu.VMEM((_CH, 8), jnp.float32),
        pltpu.VMEM((_CH, 8), jnp.float32),
        pltpu.VMEM((_CH, 8), jnp.float32),
        pltpu.VMEM((_CH,), jnp.float32),
        pltpu.VMEM((_CH,), jnp.float32),
        pltpu.VMEM((_CH,), jnp.float32),
        pltpu.VMEM((192,), jnp.float32),
        pltpu.VMEM_SHARED((_NP,), jnp.float32),
        pltpu.SemaphoreType.DMA,
        pltpu.SemaphoreType.DMA,
        pltpu.SemaphoreType.DMA,
        pltpu.SemaphoreType.DMA,
    ],
    compiler_params=pltpu.CompilerParams(
        needs_layout_passes=False, use_tc_tiling_on_sc=False),
)
def _pass2(agg0, agg1, x5, e3, wpack,
           a2o0, a2o1, v_out, inv_out, u_out,
           u_v, idx_s, idx_d, vals, p0, p1, xs, us, vs, invs, wv,
           acc, si0, si1, ss0, ss1):
    c = lax.axis_index("c")
    s = lax.axis_index("s")
    wid = s * 2 + c
    b0, nblk = _edge_split(wid)
    si = (si0, si1)
    ss = (ss0, ss1)
    ns = s * _SL

    pltpu.sync_copy(wpack, wv)
    for z in range(_CH // 16):
        us[pl.ds(z * 16, 16)] = jnp.zeros((16,), jnp.float32)
    for ci in range(_NCH):
        pltpu.sync_copy(us, acc.at[pl.ds(ns + ci * _CH, _CH)])
    lanes16 = lax.iota(jnp.int32, 16)
    wrow = [wv[pl.ds(r * 16, 16)] for r in range(12)]

    def chunk_body(ci, carry):
        base = ns + ci * _CH
        pltpu.sync_copy(agg0.at[pl.ds(base, _CH)], p0)
        pltpu.sync_copy(agg1.at[pl.ds(base, _CH)], p1)
        pltpu.sync_copy(x5.at[pl.ds(base, _CH)], xs)

        def group_body(g, carry2):
            r = g * 16 + lanes16
            k4 = jnp.full((16,), 4, jnp.int32)
            cv = plsc.load_gather(p0, [r, k4]) + plsc.load_gather(p1, [r, k4])
            inv = 1.0 / jnp.maximum(cv, 1.0)
            mean = []
            xk = []
            for k in range(4):
                kk = jnp.full((16,), k, jnp.int32)
                ak = (plsc.load_gather(p0, [r, kk])
                      + plsc.load_gather(p1, [r, kk]))
                mean.append(ak * inv)
                xk.append(plsc.load_gather(xs, [r, kk]))
            uacc = jnp.zeros((16,), jnp.float32)
            vacc = jnp.zeros((16,), jnp.float32)
            for j in range(16):
                t = jnp.broadcast_to(wrow[8][j], (16,))
                for k in range(4):
                    t = t + mean[k] * wrow[k][j] + xk[k] * wrow[4 + k][j]
                h = jnp.maximum(t, 0.0)
                uacc = uacc + h * wrow[9][j]
                vacc = vacc + h * wrow[10][j]
            us[pl.ds(g * 16, 16)] = uacc
            vs[pl.ds(g * 16, 16)] = vacc + wrow[11][0]
            invs[pl.ds(g * 16, 16)] = inv
            return carry2

        lax.fori_loop(0, _CH // 16, group_body, 0)
        pltpu.sync_copy(us, u_out.at[pl.ds(base, _CH)])
        pltpu.sync_copy(vs, v_out.at[pl.ds(base, _CH)])
        pltpu.sync_copy(invs, inv_out.at[pl.ds(base, _CH)])
        return carry

    lax.fori_loop(0, _NCH, chunk_body, 0)
    plsc.subcore_barrier()
    pltpu.sync_copy(u_out, u_v)

    def load_idx(b, blk, sem):
        r0 = (b0 + blk) * _K
        pltpu.async_copy(e3.at[0, pl.ds(r0, _K)], idx_s.at[b], sem)
        pltpu.async_copy(e3.at[1, pl.ds(r0, _K)], idx_d.at[b], sem)

    def wait_idx(b, sem):
        pltpu.make_async_copy(e3.at[0, pl.ds(0, _K)], idx_s.at[b], sem).wait()
        pltpu.make_async_copy(e3.at[1, pl.ds(0, _K)], idx_d.at[b], sem).wait()

    def compute(b):
        for j in range(_K):
            row = idx_s.at[b].at[j]
            vrow = vals.at[b].at[j]
            for k in range(_LANES // 16):
                ii = row[pl.ds(k * 16, 16)]
                vrow[pl.ds(k * 16, 16)] = plsc.load_gather(u_v, [ii])

    def fire_scatters(b, sem):
        for j in range(_K):
            pltpu.async_copy(
                vals.at[b].at[j], acc.at[idx_d.at[b].at[j]], sem, add=True)

    def wait_scatters(b, sem):
        for j in range(_K):
            pltpu.make_async_copy(
                vals.at[b].at[j], acc.at[idx_d.at[b].at[j]], sem).wait()

    load_idx(0, 0, si[0])

    def pair(p, carry):
        for b in (0, 1):
            i = 2 * p + b
            wait_idx(b, si[b])
            compute(b)
            if b == 0:
                @pl.when(p > 0)
                def _():
                    wait_scatters(1, ss[1])
            else:
                wait_scatters(0, ss[0])
            fire_scatters(b, ss[b])
            load_idx(1 - b, i + 1, si[1 - b])
        return carry

    lax.fori_loop(0, _PAIRS, pair, 0)

    wait_idx(0, si[0])
    compute(0)
    wait_scatters(1, ss[1])
    fire_scatters(0, ss[0])

    @pl.when(nblk > 97)
    def _():
        load_idx(1, 97, si[1])
        wait_idx(1, si[1])
        compute(1)
        wait_scatters(0, ss[0])
        fire_scatters(1, ss[1])
        wait_scatters(1, ss[1])

    @pl.when(nblk <= 97)
    def _():
        wait_scatters(0, ss[0])

    plsc.subcore_barrier()

    for ci in range(_NCH):
        sl_ci = pl.ds(ns + ci * _CH, _CH)
        pltpu.sync_copy(acc.at[sl_ci], us)

        @pl.when(c == 0)
        def _():
            pltpu.sync_copy(us, a2o0.at[sl_ci])

        @pl.when(c == 1)
        def _():
            pltpu.sync_copy(us, a2o1.at[sl_ci])


@functools.partial(
    pl.kernel,
    out_type=jax.ShapeDtypeStruct((_NP,), jnp.float32),
    mesh=_mesh,
    scratch_types=[
        pltpu.VMEM((_SL,), jnp.float32),
        pltpu.VMEM((_SL,), jnp.float32),
        pltpu.VMEM((_SL,), jnp.float32),
        pltpu.VMEM((_SL,), jnp.float32),
        pltpu.VMEM((_SL,), jnp.float32),
    ],
)
def _final(a2_0, a2_1, inv, v, out, a0, a1, iv, vv, ob):
    c = lax.axis_index("c")
    s = lax.axis_index("s")
    sl = s * 2 + c
    ns = sl * _SL

    @pl.when(sl < 16)
    def _():
        pltpu.sync_copy(a2_0.at[pl.ds(ns, _SL)], a0)
        pltpu.sync_copy(a2_1.at[pl.ds(ns, _SL)], a1)
        pltpu.sync_copy(inv.at[pl.ds(ns, _SL)], iv)
        pltpu.sync_copy(v.at[pl.ds(ns, _SL)], vv)

        def body(g, carry):
            sl16 = pl.ds(g * 16, 16)
            ob[sl16] = (a0[sl16] + a1[sl16]) * iv[sl16] + vv[sl16]
            return carry

        lax.fori_loop(0, _SL // 16, body, 0)
        pltpu.sync_copy(ob, out.at[pl.ds(ns, _SL)])


def kernel(x, edge_index, W1l, b1, W1r, W2l, b2, W2r):
    f32 = jnp.float32
    e3 = edge_index.reshape(2, _ROWS, _LANES)
    wpack = jnp.concatenate([
        W1l.T,
        W1r.T,
        b1.reshape(1, 16),
        W2l.reshape(1, 16),
        W2r.reshape(1, 16),
        jnp.broadcast_to(b2.reshape(1, 1), (1, 16)),
    ]).astype(f32).reshape(192)
    x5 = jnp.pad(jnp.concatenate([x, jnp.ones((_N, 1), f32)], axis=1),
                 ((0, _NP - _N), (0, 3)))
    zeros5 = jnp.zeros((_NP, 8), f32)

    agg0, agg1 = _edge_pass1(x5, e3, zeros5)
    a2_0, a2_1, v, inv, _u = _pass2(agg0, agg1, x5, e3, wpack)
    out = _final(a2_0, a2_1, inv, v)
    return out[:_N].reshape(_N, 1)
